# baseline (speedup 1.0000x reference)
import functools

import jax
import jax.numpy as jnp
from jax import lax
from jax.experimental import pallas as pl
from jax.experimental.pallas import tpu as pltpu
from jax.experimental.pallas import tpu_sc as plsc

N = 10000
E = 320000
D = 128
NC = 2
NS = 16
NW = NC * NS
EPW = E // NW
B = 128
HB = B // 2
NCH = 10
BPC = 8
NB = NCH * BPC
EP = NW * NB * B
RPT = 624
REM = N - NS * RPT


def _spmm_body(x_hbm, src_hbm, dst_hbm, vals_hbm, out_hbm,
               src_v, dst_v, vals_v, rows_v, acc_sh, sem, sem2):
    cid = lax.axis_index("c")
    sid = lax.axis_index("s")
    wid = cid * NS + sid

    zero16 = jnp.zeros((16,), jnp.float32)

    def zbody(i, c):
        r = i // 8
        col = (i % 8) * 16
        rows_v[0, r, pl.ds(col, 16)] = zero16
        return c

    lax.fori_loop(0, HB * (D // 16), zbody, 0)

    base = sid * RPT
    for k in range(RPT // HB):
        pltpu.sync_copy(rows_v.at[0], acc_sh.at[pl.ds(base + k * HB, HB)])
    rem = RPT % HB
    pltpu.sync_copy(rows_v.at[0, pl.ds(0, rem)],
                    acc_sh.at[pl.ds(base + (RPT // HB) * HB, rem)])

    @pl.when(sid == NS - 1)
    def _zero_tail():
        pltpu.sync_copy(rows_v.at[0, pl.ds(0, REM)],
                        acc_sh.at[pl.ds(NS * RPT, REM)])

    plsc.subcore_barrier()

    def scale(buf, h):
        for g in range(HB // 16):
            vv = vals_v[h, pl.ds(g * 16, 16)]
            for lane in range(16):
                e = g * 16 + lane
                v = vv[lane]
                for col in range(D // 16):
                    sl = pl.ds(col * 16, 16)
                    rows_v[buf, e, sl] = rows_v[buf, e, sl] * v

    def chunk(ch, cc):
        pltpu.sync_copy(src_hbm.at[wid, ch], src_v)
        pltpu.sync_copy(dst_hbm.at[wid, ch], dst_v)
        pltpu.sync_copy(vals_hbm.at[wid, ch], vals_v)

        def batch(j, c):
            h0 = 2 * j
            h1 = h0 + 1
            ga = pltpu.async_copy(x_hbm.at[src_v.at[h0]], rows_v.at[0],
                                  sem)
            gb = pltpu.async_copy(x_hbm.at[src_v.at[h1]], rows_v.at[1],
                                  sem)
            ga.wait()
            scale(0, h0)
            sa = pltpu.async_copy(rows_v.at[0], acc_sh.at[dst_v.at[h0]],
                                  sem2, add=True)
            gb.wait()
            scale(1, h1)
            sb = pltpu.async_copy(rows_v.at[1], acc_sh.at[dst_v.at[h1]],
                                  sem2, add=True)
            sa.wait()
            sb.wait()
            return c

        lax.fori_loop(0, BPC, batch, 0)
        return cc

    lax.fori_loop(0, NCH, chunk, 0)
    plsc.subcore_barrier()

    pltpu.sync_copy(acc_sh.at[pl.ds(base, RPT)],
                    out_hbm.at[cid, pl.ds(base, RPT)])

    @pl.when(sid == NS - 1)
    def _write_tail():
        pltpu.sync_copy(acc_sh.at[pl.ds(NS * RPT, REM)],
                        out_hbm.at[cid, pl.ds(NS * RPT, REM)])


@jax.jit
def _spmm_sc(x, src, dst, vals):
    mesh = plsc.VectorSubcoreMesh(core_axis_name="c", subcore_axis_name="s",
                                  num_cores=NC, num_subcores=NS)
    return pl.kernel(
        _spmm_body,
        out_type=jax.ShapeDtypeStruct((NC, N, D), jnp.float32),
        mesh=mesh,
        scratch_types=[
            pltpu.VMEM((2 * BPC, HB), jnp.int32),
            pltpu.VMEM((2 * BPC, HB), jnp.int32),
            pltpu.VMEM((2 * BPC, HB), jnp.float32),
            pltpu.VMEM((2, HB, D), jnp.float32),
            pltpu.VMEM_SHARED((N, D), jnp.float32),
            pltpu.SemaphoreType.DMA,
            pltpu.SemaphoreType.DMA,
        ],
    )(x, src, dst, vals)


def _merge_body(a_ref, b_ref, o_ref):
    o_ref[...] = a_ref[...] + b_ref[...]


@jax.jit
def _merge_tc(a, b):
    blk = 1000
    return pl.pallas_call(
        _merge_body,
        grid=(N // blk,),
        in_specs=[pl.BlockSpec((blk, D), lambda i: (i, 0))] * 2,
        out_specs=pl.BlockSpec((blk, D), lambda i: (i, 0)),
        out_shape=jax.ShapeDtypeStruct((N, D), jnp.float32),
    )(a, b)


def _final_body(x_ref, x0_ref, p2a_ref, p2b_ref, w0_ref, w1_ref, w2_ref,
                o_ref):
    x1 = p2a_ref[...] + p2b_ref[...]
    o_ref[...] = (
        jnp.dot(x_ref[...], w0_ref[...], preferred_element_type=jnp.float32)
        + jnp.dot(x0_ref[...], w1_ref[...], preferred_element_type=jnp.float32)
        + jnp.dot(x1, w2_ref[...], preferred_element_type=jnp.float32))


@jax.jit
def _final_tc(x, x0, p2a, p2b, w0, w1, w2):
    blk = 1000
    row = pl.BlockSpec((blk, D), lambda i: (i, 0))
    wsp = pl.BlockSpec((D, D), lambda i: (0, 0))
    return pl.pallas_call(
        _final_body,
        grid=(N // blk,),
        in_specs=[row, row, row, row, wsp, wsp, wsp],
        out_specs=row,
        out_shape=jax.ShapeDtypeStruct((N, D), jnp.float32),
    )(x, x0, p2a, p2b, w0, w1, w2)


def kernel(x, edge_index, edge_vals, W):
    pad = EP - E
    zi = jnp.zeros((pad,), jnp.int32)
    shp = (NW, NCH, 2 * BPC, HB)
    dst = jnp.concatenate([edge_index[0], zi]).reshape(shp)
    src = jnp.concatenate([edge_index[1], zi]).reshape(shp)
    vals = jnp.concatenate([edge_vals, jnp.zeros((pad,), jnp.float32)]
                           ).reshape(shp)

    p1 = _spmm_sc(x, src, dst, vals)
    x0 = _merge_tc(p1[0], p1[1])
    p2 = _spmm_sc(x0, src, dst, vals)
    y = _final_tc(x, x0, p2[0], p2[1],
                  W[:, :, 0], W[:, :, 1], W[:, :, 2])
    return y


# speedup vs baseline: 2.6274x; 2.6274x over previous
import functools

import jax
import jax.numpy as jnp
from jax import lax
from jax.experimental import pallas as pl
from jax.experimental.pallas import tpu as pltpu
from jax.experimental.pallas import tpu_sc as plsc

N = 10000
E = 320000
D = 128
NC = 2
NS = 16
NW = NC * NS
EPW = E // NW
B = 128
HB = B // 2
NCH = 10
BPC = 8
NB = NCH * BPC
EP = NW * NB * B
RPT = 624
REM = N - NS * RPT


def _spmm_body(x_hbm, src_hbm, dst_hbm, vals_hbm, out_hbm,
               src_v, dst_v, vals_v, rows_v, acc_sh, sem, sem2):
    cid = lax.axis_index("c")
    sid = lax.axis_index("s")
    wid = cid * NS + sid

    zero16 = jnp.zeros((16,), jnp.float32)

    def zbody(i, c):
        r = i // 8
        col = (i % 8) * 16
        rows_v[0, r, pl.ds(col, 16)] = zero16
        return c

    lax.fori_loop(0, HB * (D // 16), zbody, 0)

    base = sid * RPT
    for k in range(RPT // HB):
        pltpu.sync_copy(rows_v.at[0], acc_sh.at[pl.ds(base + k * HB, HB)])
    rem = RPT % HB
    pltpu.sync_copy(rows_v.at[0, pl.ds(0, rem)],
                    acc_sh.at[pl.ds(base + (RPT // HB) * HB, rem)])

    @pl.when(sid == NS - 1)
    def _zero_tail():
        pltpu.sync_copy(rows_v.at[0, pl.ds(0, REM)],
                        acc_sh.at[pl.ds(NS * RPT, REM)])

    plsc.subcore_barrier()

    def scale(buf, h):
        for g in range(HB // 16):
            vv = vals_v[h, pl.ds(g * 16, 16)]
            for lane in range(16):
                e = g * 16 + lane
                v = vv[lane]
                for col in range(D // 16):
                    sl = pl.ds(col * 16, 16)
                    rows_v[buf, e, sl] = rows_v[buf, e, sl] * v

    def chunk(ch, cc):
        pltpu.sync_copy(src_hbm.at[wid, ch], src_v)
        pltpu.sync_copy(dst_hbm.at[wid, ch], dst_v)
        pltpu.sync_copy(vals_hbm.at[wid, ch], vals_v)

        def batch(j, c):
            h0 = 2 * j
            h1 = h0 + 1
            ga = pltpu.async_copy(x_hbm.at[src_v.at[h0]], rows_v.at[0],
                                  sem)
            gb = pltpu.async_copy(x_hbm.at[src_v.at[h1]], rows_v.at[1],
                                  sem)
            ga.wait()
            scale(0, h0)
            sa = pltpu.async_copy(rows_v.at[0], acc_sh.at[dst_v.at[h0]],
                                  sem2, add=True)
            gb.wait()
            scale(1, h1)
            sb = pltpu.async_copy(rows_v.at[1], acc_sh.at[dst_v.at[h1]],
                                  sem2, add=True)
            sa.wait()
            sb.wait()
            return c

        lax.fori_loop(0, BPC, batch, 0)
        return cc

    lax.fori_loop(0, NCH, chunk, 0)
    plsc.subcore_barrier()

    pltpu.sync_copy(acc_sh.at[pl.ds(base, RPT)],
                    out_hbm.at[cid, pl.ds(base, RPT)])

    @pl.when(sid == NS - 1)
    def _write_tail():
        pltpu.sync_copy(acc_sh.at[pl.ds(NS * RPT, REM)],
                        out_hbm.at[cid, pl.ds(NS * RPT, REM)])


@jax.jit
def _spmm_sc(x, src, dst, vals):
    mesh = plsc.VectorSubcoreMesh(core_axis_name="c", subcore_axis_name="s",
                                  num_cores=NC, num_subcores=NS)
    return pl.kernel(
        _spmm_body,
        out_type=jax.ShapeDtypeStruct((NC, N, D), jnp.float32),
        mesh=mesh,
        scratch_types=[
            pltpu.VMEM((2 * BPC, HB), jnp.int32),
            pltpu.VMEM((2 * BPC, HB), jnp.int32),
            pltpu.VMEM((2 * BPC, HB), jnp.float32),
            pltpu.VMEM((2, HB, D), jnp.float32),
            pltpu.VMEM_SHARED((N, D), jnp.float32),
            pltpu.SemaphoreType.DMA,
            pltpu.SemaphoreType.DMA,
        ],
    )(x, src, dst, vals)


def _merge_body(a_ref, b_ref, o_ref):
    o_ref[...] = a_ref[...] + b_ref[...]


@jax.jit
def _merge_tc(a, b):
    blk = 1000
    return pl.pallas_call(
        _merge_body,
        grid=(N // blk,),
        in_specs=[pl.BlockSpec((blk, D), lambda i: (i, 0))] * 2,
        out_specs=pl.BlockSpec((blk, D), lambda i: (i, 0)),
        out_shape=jax.ShapeDtypeStruct((N, D), jnp.float32),
    )(a, b)


def _final_body(x_ref, x0_ref, p2a_ref, p2b_ref, w0_ref, w1_ref, w2_ref,
                o_ref):
    x1 = p2a_ref[...] + p2b_ref[...]
    o_ref[...] = (
        jnp.dot(x_ref[...], w0_ref[...], preferred_element_type=jnp.float32)
        + jnp.dot(x0_ref[...], w1_ref[...], preferred_element_type=jnp.float32)
        + jnp.dot(x1, w2_ref[...], preferred_element_type=jnp.float32))


@jax.jit
def _final_tc(x, x0, p2a, p2b, w0, w1, w2):
    blk = 1000
    row = pl.BlockSpec((blk, D), lambda i: (i, 0))
    wsp = pl.BlockSpec((D, D), lambda i: (0, 0))
    return pl.pallas_call(
        _final_body,
        grid=(N // blk,),
        in_specs=[row, row, row, row, wsp, wsp, wsp],
        out_specs=row,
        out_shape=jax.ShapeDtypeStruct((N, D), jnp.float32),
    )(x, x0, p2a, p2b, w0, w1, w2)


def kernel(x, edge_index, edge_vals, W):
    pad = EP - E
    zi = jnp.arange(pad, dtype=jnp.int32) % N
    shp = (NW, NCH, 2 * BPC, HB)
    dst = jnp.concatenate([edge_index[0], zi]).reshape(shp)
    src = jnp.concatenate([edge_index[1], zi]).reshape(shp)
    vals = jnp.concatenate([edge_vals, jnp.zeros((pad,), jnp.float32)]
                           ).reshape(shp)

    p1 = _spmm_sc(x, src, dst, vals)
    x0 = _merge_tc(p1[0], p1[1])
    p2 = _spmm_sc(x0, src, dst, vals)
    y = _final_tc(x, x0, p2[0], p2[1],
                  W[:, :, 0], W[:, :, 1], W[:, :, 2])
    return y
